# baseline (speedup 1.0000x reference)
import jax
import jax.numpy as jnp
from jax.experimental import pallas as pl
from jax.experimental.pallas import tpu as pltpu

EPS = 1e-12
BLOCK_R = 2048


NSUB = 4


def _body(x_ref, pos_ref, g_ref, b_ref, o_ref):
    rows_per_s = pos_ref.shape[0] // BLOCK_R
    i = pl.program_id(0) % rows_per_s
    sub = BLOCK_R // NSUB
    g = g_ref[...]
    b = b_ref[...]
    for c in range(NSUB):
        r0 = c * sub
        p = pos_ref[pl.ds(i * BLOCK_R + r0, sub), :]
        h = x_ref[pl.ds(r0, sub), :] + p
        inv_d = 1.0 / h.shape[-1]
        ones = jnp.ones((h.shape[-1], 1), jnp.float32)
        mean = jax.lax.dot_general(
            h, ones, (((1,), (0,)), ((), ())),
            preferred_element_type=jnp.float32,
        ) * inv_d
        ex2 = jax.lax.dot_general(
            h * h, ones, (((1,), (0,)), ((), ())),
            preferred_element_type=jnp.float32,
        ) * inv_d
        var = ex2 - mean * mean
        k = jax.lax.rsqrt(var + EPS)
        o_ref[pl.ds(r0, sub), :] = ((h - mean) * k) * g + b


def kernel(x, pos_table, ln_gamma, ln_beta):
    B, S, D = x.shape
    rows = B * S
    out = pl.pallas_call(
        _body,
        grid=(rows // BLOCK_R,),
        in_specs=[
            pl.BlockSpec((BLOCK_R, D), lambda i: (i, 0)),
            pl.BlockSpec((S, D), lambda i: (0, 0)),
            pl.BlockSpec((D,), lambda i: (0,)),
            pl.BlockSpec((D,), lambda i: (0,)),
        ],
        out_specs=pl.BlockSpec((BLOCK_R, D), lambda i: (i, 0)),
        out_shape=jax.ShapeDtypeStruct((rows, D), x.dtype),
        compiler_params=pltpu.CompilerParams(
            vmem_limit_bytes=100 * 1024 * 1024,
        ),
    )(x.reshape(rows, D), pos_table, ln_gamma, ln_beta)
    return out.reshape(B, S, D)


# speedup vs baseline: 1.0301x; 1.0301x over previous
import jax
import jax.numpy as jnp
from jax.experimental import pallas as pl

EPS = 1e-12
BLOCK_R = 2048


NSUB = 4


def _body(x_ref, pos_ref, g_ref, b_ref, o_ref):
    rows_per_s = pos_ref.shape[0] // BLOCK_R
    i = pl.program_id(0) % rows_per_s
    sub = BLOCK_R // NSUB
    g = g_ref[...]
    b = b_ref[...]
    for c in range(NSUB):
        r0 = c * sub
        p = pos_ref[pl.ds(i * BLOCK_R + r0, sub), :]
        h = x_ref[pl.ds(r0, sub), :] + p
        inv_d = 1.0 / h.shape[-1]
        mean = jnp.sum(h, axis=-1, keepdims=True) * inv_d
        ex2 = jnp.sum(h * h, axis=-1, keepdims=True) * inv_d
        var = ex2 - mean * mean
        k = jax.lax.rsqrt(var + EPS)
        o_ref[pl.ds(r0, sub), :] = ((h - mean) * k) * g + b


def kernel(x, pos_table, ln_gamma, ln_beta):
    B, S, D = x.shape
    rows = B * S
    out = pl.pallas_call(
        _body,
        grid=(rows // BLOCK_R,),
        in_specs=[
            pl.BlockSpec((BLOCK_R, D), lambda i: (i, 0)),
            pl.BlockSpec((S, D), lambda i: (0, 0)),
            pl.BlockSpec((D,), lambda i: (0,)),
            pl.BlockSpec((D,), lambda i: (0,)),
        ],
        out_specs=pl.BlockSpec((BLOCK_R, D), lambda i: (i, 0)),
        out_shape=jax.ShapeDtypeStruct((rows, D), x.dtype),
    )(x.reshape(rows, D), pos_table, ln_gamma, ln_beta)
    return out.reshape(B, S, D)
